# Initial kernel scaffold; baseline (speedup 1.0000x reference)
import jax
import jax.numpy as jnp
from jax.experimental import pallas as pl


def kernel(x, edge_index, edge_type, W1, root1, b1, W2, root2, b2):
    raise NotImplementedError("write your pallas kernel here")


# speedup vs baseline: 10.9810x; 10.9810x over previous
import functools

import jax
import jax.numpy as jnp
from jax import lax
from jax.experimental import pallas as pl
from jax.experimental.pallas import tpu as pltpu
from jax.experimental.pallas import tpu_sc as plsc

N = 10000
E = 320000
D = 128
R = 8
RN = R * N
NC = 2
NS = 16
NW = NC * NS
CH = 80
EC = E // NW
ECC = E // NS
NPT = N // NS
CPT = RN // NS
ZR = 125
NB = 10
BN = N // NB

_mesh = plsc.VectorSubcoreMesh(
    core_axis_name="c", subcore_axis_name="s", num_cores=NC, num_subcores=NS)


@functools.partial(
    pl.kernel,
    out_type=(jax.ShapeDtypeStruct((E,), jnp.int32),
              jax.ShapeDtypeStruct((E,), jnp.float32)),
    mesh=_mesh,
    compiler_params=pltpu.CompilerParams(needs_layout_passes=False, use_tc_tiling_on_sc=False),
    scratch_types=[
        pltpu.VMEM_SHARED((RN,), jnp.float32),
        pltpu.VMEM((RN,), jnp.float32),
        pltpu.VMEM((CH,), jnp.int32),
        pltpu.VMEM((CH,), jnp.int32),
        pltpu.VMEM((CH,), jnp.int32),
        pltpu.VMEM((CH,), jnp.int32),
        pltpu.VMEM((CH,), jnp.float32),
        pltpu.VMEM((CH,), jnp.int32),
        pltpu.VMEM((CH,), jnp.float32),
    ],
)
def _prep(et_h, src_h, dst_h, gidx_h, scale_h,
          counts_sp, invb, etb, srcb, dstb, sidxb, onesb, gidxb, scaleb):
    c = lax.axis_index("c")
    s = lax.axis_index("s")
    w = s * NC + c

    for i in range(CH // 16):
        onesb[pl.ds(i * 16, 16)] = jnp.ones((16,), jnp.float32)

    def zbody(i, carry):
        invb[pl.ds(i * 16, 16)] = jnp.zeros((16,), jnp.float32)
        return carry

    lax.fori_loop(0, (CPT + 15) // 16, zbody, 0)
    pltpu.sync_copy(invb.at[pl.ds(0, CPT)], counts_sp.at[pl.ds(s * CPT, CPT)])
    plsc.subcore_barrier()

    cbase = s * ECC

    def cbody(k, carry):
        off = cbase + k * CH
        pltpu.sync_copy(et_h.at[pl.ds(off, CH)], etb)
        pltpu.sync_copy(dst_h.at[pl.ds(off, CH)], dstb)
        for i in range(CH // 16):
            sl = pl.ds(i * 16, 16)
            sidxb[sl] = etb[sl] * N + dstb[sl]
        pltpu.sync_copy(onesb, counts_sp.at[sidxb], add=True)
        return carry

    lax.fori_loop(0, ECC // CH, cbody, 0)
    plsc.subcore_barrier()

    tb = s * CPT
    pltpu.sync_copy(counts_sp.at[pl.ds(tb, CPT)], invb.at[pl.ds(0, CPT)])

    def ibody(i, carry):
        sl = pl.ds(i * 16, 16)
        invb[sl] = 1.0 / jnp.maximum(invb[sl], 1.0)
        return carry

    lax.fori_loop(0, (CPT + 15) // 16, ibody, 0)
    pltpu.sync_copy(invb.at[pl.ds(0, CPT)], counts_sp.at[pl.ds(tb, CPT)])
    plsc.subcore_barrier()

    pltpu.sync_copy(counts_sp, invb)

    ebase = w * EC

    def ebody(k, carry):
        off = ebase + k * CH
        pltpu.sync_copy(et_h.at[pl.ds(off, CH)], etb)
        pltpu.sync_copy(src_h.at[pl.ds(off, CH)], srcb)
        pltpu.sync_copy(dst_h.at[pl.ds(off, CH)], dstb)
        for i in range(CH // 16):
            sl = pl.ds(i * 16, 16)
            et16 = etb[sl]
            gidxb[sl] = et16 * N + srcb[sl]
            scaleb[sl] = plsc.load_gather(invb, [et16 * N + dstb[sl]])
        pltpu.sync_copy(gidxb, gidx_h.at[pl.ds(off, CH)])
        pltpu.sync_copy(scaleb, scale_h.at[pl.ds(off, CH)])
        return carry

    lax.fori_loop(0, EC // CH, ebody, 0)


@functools.partial(
    pl.kernel,
    out_type=jax.ShapeDtypeStruct((NC * N, D), jnp.float32),
    mesh=_mesh,
    compiler_params=pltpu.CompilerParams(needs_layout_passes=False, use_tc_tiling_on_sc=False),
    scratch_types=[
        pltpu.VMEM_SHARED((N, D), jnp.float32),
        pltpu.VMEM((ZR, D), jnp.float32),
        pltpu.VMEM((CH,), jnp.int32),
        pltpu.VMEM((CH,), jnp.float32),
        pltpu.VMEM((CH,), jnp.int32),
        pltpu.VMEM((CH, D), jnp.float32),
        pltpu.SemaphoreType.DMA,
    ],
)
def _edge_pass(xw_h, gidx_h, scale_h, dst_h, aggs_h,
               acc_sp, zrows, gixb, scb, dstb, rows, sem):
    c = lax.axis_index("c")
    s = lax.axis_index("s")
    w = s * NC + c

    def zfill(r, carry):
        for i in range(D // 16):
            zrows[r, pl.ds(i * 16, 16)] = jnp.zeros((16,), jnp.float32)
        return carry

    lax.fori_loop(0, ZR, zfill, 0)
    for m in range(NPT // ZR):
        pltpu.sync_copy(zrows, acc_sp.at[pl.ds(s * NPT + m * ZR, ZR)])
    plsc.subcore_barrier()

    ebase = w * EC

    def ebody(k, carry):
        off = ebase + k * CH
        pltpu.sync_copy(gidx_h.at[pl.ds(off, CH)], gixb)
        pltpu.sync_copy(scale_h.at[pl.ds(off, CH)], scb)
        pltpu.sync_copy(dst_h.at[pl.ds(off, CH)], dstb)
        pltpu.async_copy(xw_h.at[gixb], rows, sem).wait()

        def mbody(j, mcarry):
            sp = plsc.load_gather(scb, [jnp.full((16,), j, jnp.int32)])
            for i in range(D // 16):
                sl = pl.ds(i * 16, 16)
                rows[j, sl] = rows[j, sl] * sp
            return mcarry

        lax.fori_loop(0, CH, mbody, 0)
        pltpu.sync_copy(rows, acc_sp.at[dstb], add=True)
        return carry

    lax.fori_loop(0, EC // CH, ebody, 0)
    plsc.subcore_barrier()
    pltpu.sync_copy(acc_sp.at[pl.ds(s * NPT, NPT)],
                    aggs_h.at[pl.ds(c * N + s * NPT, NPT)])


def _xw_body(x_ref, w_ref, o_ref):
    o_ref[...] = jnp.dot(x_ref[...], w_ref[0],
                         preferred_element_type=jnp.float32)


def _tc_xw(act, W):
    return pl.pallas_call(
        _xw_body,
        grid=(R, NB),
        in_specs=[pl.BlockSpec((BN, D), lambda r, n: (n, 0)),
                  pl.BlockSpec((1, D, D), lambda r, n: (r, 0, 0))],
        out_specs=pl.BlockSpec((BN, D), lambda r, n: (r * NB + n, 0)),
        out_shape=jax.ShapeDtypeStruct((RN, D), jnp.float32),
    )(act, W)


def _combine_body(relu, a0_ref, a1_ref, act_ref, root_ref, b_ref, o_ref):
    h = (a0_ref[...] + a1_ref[...] + b_ref[...]
         + jnp.dot(act_ref[...], root_ref[...],
                   preferred_element_type=jnp.float32))
    if relu:
        h = jnp.maximum(h, 0.0)
    o_ref[...] = h


def _tc_combine(aggs, act, root, b, relu):
    return pl.pallas_call(
        functools.partial(_combine_body, relu),
        grid=(NB,),
        in_specs=[pl.BlockSpec((BN, D), lambda n: (n, 0)),
                  pl.BlockSpec((BN, D), lambda n: (NB + n, 0)),
                  pl.BlockSpec((BN, D), lambda n: (n, 0)),
                  pl.BlockSpec((D, D), lambda n: (0, 0)),
                  pl.BlockSpec((1, D), lambda n: (0, 0))],
        out_specs=pl.BlockSpec((BN, D), lambda n: (n, 0)),
        out_shape=jax.ShapeDtypeStruct((N, D), jnp.float32),
    )(aggs, aggs, act, root, b.reshape(1, D))


def kernel(x, edge_index, edge_type, W1, root1, b1, W2, root2, b2):
    src = edge_index[0]
    dst = edge_index[1]
    gidx, scale = _prep(edge_type, src, dst)
    h = x
    for (W, root, b, relu) in ((W1, root1, b1, True), (W2, root2, b2, False)):
        xw = _tc_xw(h, W)
        aggs = _edge_pass(xw, gidx, scale, dst)
        h = _tc_combine(aggs, h, root, b, relu)
    return h


# speedup vs baseline: 17.9981x; 1.6390x over previous
import functools

import jax
import jax.numpy as jnp
from jax import lax
from jax.experimental import pallas as pl
from jax.experimental.pallas import tpu as pltpu
from jax.experimental.pallas import tpu_sc as plsc

N = 10000
E = 320000
D = 128
R = 8
RN = R * N
NC = 2
NS = 16
NW = NC * NS
CH = 80
EC = E // NW
ECC = E // NS
NPT = N // NS
CPT = RN // NS
ZR = 125
NB = 10
BN = N // NB

_mesh = plsc.VectorSubcoreMesh(
    core_axis_name="c", subcore_axis_name="s", num_cores=NC, num_subcores=NS)


@functools.partial(
    pl.kernel,
    out_type=(jax.ShapeDtypeStruct((E,), jnp.int32),
              jax.ShapeDtypeStruct((E,), jnp.float32)),
    mesh=_mesh,
    compiler_params=pltpu.CompilerParams(needs_layout_passes=False, use_tc_tiling_on_sc=False),
    scratch_types=[
        pltpu.VMEM_SHARED((RN,), jnp.float32),
        pltpu.VMEM((RN,), jnp.float32),
        pltpu.VMEM((CH,), jnp.int32),
        pltpu.VMEM((CH,), jnp.int32),
        pltpu.VMEM((CH,), jnp.int32),
        pltpu.VMEM((CH,), jnp.int32),
        pltpu.VMEM((CH,), jnp.float32),
        pltpu.VMEM((CH,), jnp.int32),
        pltpu.VMEM((CH,), jnp.float32),
    ],
)
def _prep(et_h, src_h, dst_h, gidx_h, scale_h,
          counts_sp, invb, etb, srcb, dstb, sidxb, onesb, gidxb, scaleb):
    c = lax.axis_index("c")
    s = lax.axis_index("s")
    w = s * NC + c

    for i in range(CH // 16):
        onesb[pl.ds(i * 16, 16)] = jnp.ones((16,), jnp.float32)

    def zbody(i, carry):
        invb[pl.ds(i * 16, 16)] = jnp.zeros((16,), jnp.float32)
        return carry

    lax.fori_loop(0, (CPT + 15) // 16, zbody, 0)
    pltpu.sync_copy(invb.at[pl.ds(0, CPT)], counts_sp.at[pl.ds(s * CPT, CPT)])
    plsc.subcore_barrier()

    cbase = s * ECC

    def cbody(k, carry):
        off = cbase + k * CH
        pltpu.sync_copy(et_h.at[pl.ds(off, CH)], etb)
        pltpu.sync_copy(dst_h.at[pl.ds(off, CH)], dstb)
        for i in range(CH // 16):
            sl = pl.ds(i * 16, 16)
            sidxb[sl] = etb[sl] * N + dstb[sl]
        pltpu.sync_copy(onesb, counts_sp.at[sidxb], add=True)
        return carry

    lax.fori_loop(0, ECC // CH, cbody, 0)
    plsc.subcore_barrier()

    tb = s * CPT
    pltpu.sync_copy(counts_sp.at[pl.ds(tb, CPT)], invb.at[pl.ds(0, CPT)])

    def ibody(i, carry):
        sl = pl.ds(i * 16, 16)
        invb[sl] = 1.0 / jnp.maximum(invb[sl], 1.0)
        return carry

    lax.fori_loop(0, (CPT + 15) // 16, ibody, 0)
    pltpu.sync_copy(invb.at[pl.ds(0, CPT)], counts_sp.at[pl.ds(tb, CPT)])
    plsc.subcore_barrier()

    pltpu.sync_copy(counts_sp, invb)

    ebase = w * EC

    def ebody(k, carry):
        off = ebase + k * CH
        pltpu.sync_copy(et_h.at[pl.ds(off, CH)], etb)
        pltpu.sync_copy(src_h.at[pl.ds(off, CH)], srcb)
        pltpu.sync_copy(dst_h.at[pl.ds(off, CH)], dstb)
        for i in range(CH // 16):
            sl = pl.ds(i * 16, 16)
            et16 = etb[sl]
            gidxb[sl] = et16 * N + srcb[sl]
            scaleb[sl] = plsc.load_gather(invb, [et16 * N + dstb[sl]])
        pltpu.sync_copy(gidxb, gidx_h.at[pl.ds(off, CH)])
        pltpu.sync_copy(scaleb, scale_h.at[pl.ds(off, CH)])
        return carry

    lax.fori_loop(0, EC // CH, ebody, 0)


CHE = 100
NCHE = EC // CHE


@functools.partial(
    pl.kernel,
    out_type=jax.ShapeDtypeStruct((NC * N, D), jnp.float32),
    mesh=_mesh,
    compiler_params=pltpu.CompilerParams(needs_layout_passes=False, use_tc_tiling_on_sc=False),
    scratch_types=[
        pltpu.VMEM_SHARED((N, D), jnp.float32),
        pltpu.VMEM((25, D), jnp.float32),
        pltpu.VMEM((NCHE, CHE), jnp.int32),
        pltpu.VMEM((EC,), jnp.float32),
        pltpu.VMEM((CHE,), jnp.int32),
        pltpu.VMEM((CHE,), jnp.int32),
        pltpu.VMEM((CHE, D), jnp.float32),
        pltpu.VMEM((CHE, D), jnp.float32),
        pltpu.SemaphoreType.DMA,
        pltpu.SemaphoreType.DMA,
        pltpu.SemaphoreType.DMA,
        pltpu.SemaphoreType.DMA,
        pltpu.SemaphoreType.DMA,
        pltpu.SemaphoreType.DMA,
        pltpu.SemaphoreType.DMA,
    ],
)
def _edge_pass(xw_h, gidx2_h, scale_h, dst2_h, aggs_h,
               acc_sp, zrows, gidxv, scalev, dstb0, dstb1,
               rows0, rows1, sz, sg0, sg1, ss0, ss1, sd0, sd1):
    c = lax.axis_index("c")
    s = lax.axis_index("s")
    w = s * NC + c

    pltpu.sync_copy(gidx2_h.at[pl.ds(w * NCHE, NCHE)], gidxv)
    pltpu.sync_copy(scale_h.at[pl.ds(w * EC, EC)], scalev)

    def zfill(r, carry):
        for i in range(D // 16):
            zrows[r, pl.ds(i * 16, 16)] = jnp.zeros((16,), jnp.float32)
        return carry

    lax.fori_loop(0, 25, zfill, 0)
    for m in range(NPT // 25):
        pltpu.async_copy(zrows, acc_sp.at[pl.ds(s * NPT + m * 25, 25)], sz)
    for m in range(NPT // 25):
        pltpu.make_async_copy(zrows, acc_sp.at[pl.ds(s * NPT + m * 25, 25)],
                              sz).wait()
    plsc.subcore_barrier()

    bufs = ((rows0, sg0, ss0, dstb0, sd0), (rows1, sg1, ss1, dstb1, sd1))
    row0 = w * NCHE

    pltpu.async_copy(xw_h.at[gidxv.at[0]], rows0, sg0)
    pltpu.async_copy(dst2_h.at[row0], dstb0, sd0)

    def pipe(i, carry):
        for b in range(2):
            rows, sg, ss, dstb, sd = bufs[b]
            rows_o, sg_o, ss_o, dstb_o, sd_o = bufs[1 - b]
            k = i * 2 + b

            @pl.when(k >= 1)
            def _():
                pltpu.make_async_copy(
                    rows_o, acc_sp.at[dstb_o], ss_o).wait()

            @pl.when(k + 1 < NCHE)
            def _():
                pltpu.async_copy(xw_h.at[gidxv.at[k + 1]], rows_o, sg_o)
                pltpu.async_copy(dst2_h.at[row0 + k + 1], dstb_o, sd_o)

            pltpu.make_async_copy(xw_h.at[gidxv.at[k]], rows, sg).wait()
            pltpu.make_async_copy(dst2_h.at[row0 + k], dstb, sd).wait()

            def mbody(j, mcarry):
                sp = plsc.load_gather(
                    scalev, [jnp.full((16,), k * CHE + j, jnp.int32)])
                for i2 in range(D // 16):
                    sl = pl.ds(i2 * 16, 16)
                    rows[j, sl] = rows[j, sl] * sp
                return mcarry

            lax.fori_loop(0, CHE, mbody, 0)
            pltpu.async_copy(rows, acc_sp.at[dstb], ss, add=True)
        return carry

    lax.fori_loop(0, NCHE // 2, pipe, 0)
    pltpu.make_async_copy(rows1, acc_sp.at[dstb1], ss1).wait()
    plsc.subcore_barrier()
    pltpu.sync_copy(acc_sp.at[pl.ds(s * NPT, NPT)],
                    aggs_h.at[pl.ds(c * N + s * NPT, NPT)])


def _xw_body(x_ref, w_ref, o_ref):
    o_ref[...] = jnp.dot(x_ref[...], w_ref[0],
                         preferred_element_type=jnp.float32)


def _tc_xw(act, W):
    return pl.pallas_call(
        _xw_body,
        grid=(R, NB),
        in_specs=[pl.BlockSpec((BN, D), lambda r, n: (n, 0)),
                  pl.BlockSpec((1, D, D), lambda r, n: (r, 0, 0))],
        out_specs=pl.BlockSpec((BN, D), lambda r, n: (r * NB + n, 0)),
        out_shape=jax.ShapeDtypeStruct((RN, D), jnp.float32),
    )(act, W)


def _combine_body(relu, a0_ref, a1_ref, act_ref, root_ref, b_ref, o_ref):
    h = (a0_ref[...] + a1_ref[...] + b_ref[...]
         + jnp.dot(act_ref[...], root_ref[...],
                   preferred_element_type=jnp.float32))
    if relu:
        h = jnp.maximum(h, 0.0)
    o_ref[...] = h


def _tc_combine(aggs, act, root, b, relu):
    return pl.pallas_call(
        functools.partial(_combine_body, relu),
        grid=(NB,),
        in_specs=[pl.BlockSpec((BN, D), lambda n: (n, 0)),
                  pl.BlockSpec((BN, D), lambda n: (NB + n, 0)),
                  pl.BlockSpec((BN, D), lambda n: (n, 0)),
                  pl.BlockSpec((D, D), lambda n: (0, 0)),
                  pl.BlockSpec((1, D), lambda n: (0, 0))],
        out_specs=pl.BlockSpec((BN, D), lambda n: (n, 0)),
        out_shape=jax.ShapeDtypeStruct((N, D), jnp.float32),
    )(aggs, aggs, act, root, b.reshape(1, D))


def kernel(x, edge_index, edge_type, W1, root1, b1, W2, root2, b2):
    src = edge_index[0]
    dst = edge_index[1]
    gidx, scale = _prep(edge_type, src, dst)
    gidx2 = gidx.reshape(E // CHE, CHE)
    dst2 = dst.reshape(E // CHE, CHE)
    h = x
    for (W, root, b, relu) in ((W1, root1, b1, True), (W2, root2, b2, False)):
        xw = _tc_xw(h, W)
        aggs = _edge_pass(xw, gidx2, scale, dst2)
        h = _tc_combine(aggs, h, root, b, relu)
    return h


# speedup vs baseline: 21.9111x; 1.2174x over previous
import functools

import jax
import jax.numpy as jnp
from jax import lax
from jax.experimental import pallas as pl
from jax.experimental.pallas import tpu as pltpu
from jax.experimental.pallas import tpu_sc as plsc

N = 10000
E = 320000
D = 128
R = 8
RN = R * N
NC = 2
NS = 16
NW = NC * NS
CH = 80
EC = E // NW
ECC = E // NS
NPT = N // NS
CPT = RN // NS
ZR = 125
NB = 10
BN = N // NB

_mesh = plsc.VectorSubcoreMesh(
    core_axis_name="c", subcore_axis_name="s", num_cores=NC, num_subcores=NS)


NCHO = EC // CH
CE = 2000
NSC = E // CE


@functools.partial(
    pl.kernel,
    out_type=(jax.ShapeDtypeStruct((E // CH, CH), jnp.int32),
              jax.ShapeDtypeStruct((E // CH, CH), jnp.float32),
              jax.ShapeDtypeStruct((RN,), jnp.float32)),
    mesh=_mesh,
    compiler_params=pltpu.CompilerParams(needs_layout_passes=False, use_tc_tiling_on_sc=False),
    scratch_types=[
        pltpu.VMEM((CPT + 16,), jnp.float32),
        pltpu.VMEM((CE,), jnp.int32),
        pltpu.VMEM((CE,), jnp.int32),
        pltpu.VMEM((CE,), jnp.int32),
        pltpu.VMEM((CE,), jnp.int32),
        pltpu.VMEM((EC,), jnp.int32),
        pltpu.VMEM((EC,), jnp.int32),
        pltpu.VMEM((EC,), jnp.int32),
        pltpu.VMEM((NCHO, CH), jnp.int32),
        pltpu.VMEM((NCHO, CH), jnp.float32),
        pltpu.SemaphoreType.DMA,
        pltpu.SemaphoreType.DMA,
        pltpu.SemaphoreType.DMA,
    ],
)
def _prep(et_h, src_h, dst_h, gidx_h, scale_h, inv_h,
          hist, etc0, etc1, dstc0, dstc1, etv, tmpv, sidxv, gidx2, scale2,
          sp0, sp1, sg_sem):
    c = lax.axis_index("c")
    s = lax.axis_index("s")
    w = s * NC + c
    lo = s * CPT

    def zbody(i, carry):
        hist[pl.ds(i * 16, 16)] = jnp.zeros((16,), jnp.float32)
        return carry

    lax.fori_loop(0, (CPT + 15) // 16, zbody, 0)

    obase = w * EC
    pltpu.sync_copy(et_h.at[pl.ds(obase, EC)], etv)
    pltpu.sync_copy(dst_h.at[pl.ds(obase, EC)], tmpv)

    ones16 = jnp.ones((16,), jnp.float32)
    bufs = ((etc0, dstc0, sp0), (etc1, dstc1, sp1))

    pltpu.async_copy(et_h.at[pl.ds(0, CE)], etc0, sp0)
    pltpu.async_copy(dst_h.at[pl.ds(0, CE)], dstc0, sp0)

    def scan(i, carry):
        for b in range(2):
            etc, dstc, sp = bufs[b]
            etc_o, dstc_o, sp_o = bufs[1 - b]
            k = i * 2 + b

            @pl.when(k + 1 < NSC)
            def _():
                off = (k + 1) * CE
                pltpu.async_copy(et_h.at[pl.ds(off, CE)], etc_o, sp_o)
                pltpu.async_copy(dst_h.at[pl.ds(off, CE)], dstc_o, sp_o)

            off = k * CE
            pltpu.make_async_copy(et_h.at[pl.ds(off, CE)], etc, sp).wait()
            pltpu.make_async_copy(dst_h.at[pl.ds(off, CE)], dstc, sp).wait()

            def hbody(j, hcarry):
                sl = pl.ds(j * 16, 16)
                sidx = etc[sl] * N + dstc[sl]
                li = sidx - lo
                m = (li >= 0) & (li < CPT)
                li = jnp.minimum(jnp.maximum(li, 0), CPT - 1)
                plsc.addupdate_scatter(hist, [li], ones16, mask=m)
                return hcarry

            lax.fori_loop(0, CE // 16, hbody, 0)
        return carry

    lax.fori_loop(0, NSC // 2, scan, 0)

    def ibody(i, carry):
        sl = pl.ds(i * 16, 16)
        hist[sl] = 1.0 / jnp.maximum(hist[sl], 1.0)
        return carry

    lax.fori_loop(0, (CPT + 15) // 16, ibody, 0)
    pltpu.sync_copy(hist.at[pl.ds(0, CPT)], inv_h.at[pl.ds(lo, CPT)])

    def sbody(r, carry):
        for i in range(CH // 16):
            sl = pl.ds(r * CH + i * 16, 16)
            sidxv[sl] = etv[sl] * N + tmpv[sl]
        return carry

    lax.fori_loop(0, NCHO, sbody, 0)

    pltpu.sync_copy(src_h.at[pl.ds(obase, EC)], tmpv)

    def gbody(r, carry):
        for i in range(CH // 16):
            sl = pl.ds(r * CH + i * 16, 16)
            gidx2[r, pl.ds(i * 16, 16)] = etv[sl] * N + tmpv[sl]
        return carry

    lax.fori_loop(0, NCHO, gbody, 0)
    plsc.subcore_barrier()

    def gfire(k, carry):
        idx = sidxv.at[pl.ds(k * CH, CH)]
        pltpu.async_copy(inv_h.at[idx], scale2.at[k], sg_sem)
        return carry

    lax.fori_loop(0, NCHO, gfire, 0)

    def gdrain(k, carry):
        idx = sidxv.at[pl.ds(k * CH, CH)]
        pltpu.make_async_copy(inv_h.at[idx], scale2.at[k], sg_sem).wait()
        return carry

    lax.fori_loop(0, NCHO, gdrain, 0)
    pltpu.sync_copy(gidx2, gidx_h.at[pl.ds(w * NCHO, NCHO)])
    pltpu.sync_copy(scale2, scale_h.at[pl.ds(w * NCHO, NCHO)])


CHE = 100
NCHE = EC // CHE


@functools.partial(
    pl.kernel,
    out_type=jax.ShapeDtypeStruct((NC * N, D), jnp.float32),
    mesh=_mesh,
    compiler_params=pltpu.CompilerParams(needs_layout_passes=False, use_tc_tiling_on_sc=False),
    scratch_types=[
        pltpu.VMEM_SHARED((N, D), jnp.float32),
        pltpu.VMEM((25, D), jnp.float32),
        pltpu.VMEM((NCHE, CHE), jnp.int32),
        pltpu.VMEM((EC,), jnp.float32),
        pltpu.VMEM((CHE,), jnp.int32),
        pltpu.VMEM((CHE,), jnp.int32),
        pltpu.VMEM((CHE, D), jnp.float32),
        pltpu.VMEM((CHE, D), jnp.float32),
        pltpu.SemaphoreType.DMA,
        pltpu.SemaphoreType.DMA,
        pltpu.SemaphoreType.DMA,
        pltpu.SemaphoreType.DMA,
        pltpu.SemaphoreType.DMA,
        pltpu.SemaphoreType.DMA,
        pltpu.SemaphoreType.DMA,
    ],
)
def _edge_pass(xw_h, gidx2_h, scale_h, dst2_h, aggs_h,
               acc_sp, zrows, gidxv, scalev, dstb0, dstb1,
               rows0, rows1, sz, sg0, sg1, ss0, ss1, sd0, sd1):
    c = lax.axis_index("c")
    s = lax.axis_index("s")
    w = s * NC + c

    pltpu.sync_copy(gidx2_h.at[pl.ds(w * NCHE, NCHE)], gidxv)
    pltpu.sync_copy(scale_h.at[pl.ds(w * EC, EC)], scalev)

    def zfill(r, carry):
        for i in range(D // 16):
            zrows[r, pl.ds(i * 16, 16)] = jnp.zeros((16,), jnp.float32)
        return carry

    lax.fori_loop(0, 25, zfill, 0)
    for m in range(NPT // 25):
        pltpu.async_copy(zrows, acc_sp.at[pl.ds(s * NPT + m * 25, 25)], sz)
    for m in range(NPT // 25):
        pltpu.make_async_copy(zrows, acc_sp.at[pl.ds(s * NPT + m * 25, 25)],
                              sz).wait()
    plsc.subcore_barrier()

    bufs = ((rows0, sg0, ss0, dstb0, sd0), (rows1, sg1, ss1, dstb1, sd1))
    row0 = w * NCHE

    pltpu.async_copy(xw_h.at[gidxv.at[0]], rows0, sg0)
    pltpu.async_copy(dst2_h.at[row0], dstb0, sd0)

    def pipe(i, carry):
        for b in range(2):
            rows, sg, ss, dstb, sd = bufs[b]
            rows_o, sg_o, ss_o, dstb_o, sd_o = bufs[1 - b]
            k = i * 2 + b

            @pl.when(k >= 1)
            def _():
                pltpu.make_async_copy(
                    rows_o, acc_sp.at[dstb_o], ss_o).wait()

            @pl.when(k + 1 < NCHE)
            def _():
                pltpu.async_copy(xw_h.at[gidxv.at[k + 1]], rows_o, sg_o)
                pltpu.async_copy(dst2_h.at[row0 + k + 1], dstb_o, sd_o)

            pltpu.make_async_copy(xw_h.at[gidxv.at[k]], rows, sg).wait()
            pltpu.make_async_copy(dst2_h.at[row0 + k], dstb, sd).wait()

            def mbody(j, mcarry):
                sp = plsc.load_gather(
                    scalev, [jnp.full((16,), k * CHE + j, jnp.int32)])
                for i2 in range(D // 16):
                    sl = pl.ds(i2 * 16, 16)
                    rows[j, sl] = rows[j, sl] * sp
                return mcarry

            lax.fori_loop(0, CHE, mbody, 0)
            pltpu.async_copy(rows, acc_sp.at[dstb], ss, add=True)
        return carry

    lax.fori_loop(0, NCHE // 2, pipe, 0)
    pltpu.make_async_copy(rows1, acc_sp.at[dstb1], ss1).wait()
    plsc.subcore_barrier()
    pltpu.sync_copy(acc_sp.at[pl.ds(s * NPT, NPT)],
                    aggs_h.at[pl.ds(c * N + s * NPT, NPT)])


def _xw_body(x_ref, w_ref, o_ref):
    o_ref[...] = jnp.dot(x_ref[...], w_ref[0],
                         preferred_element_type=jnp.float32,
                         precision=jax.lax.Precision.HIGHEST)


def _tc_xw(act, W):
    return pl.pallas_call(
        _xw_body,
        grid=(R, NB),
        in_specs=[pl.BlockSpec((BN, D), lambda r, n: (n, 0)),
                  pl.BlockSpec((1, D, D), lambda r, n: (r, 0, 0))],
        out_specs=pl.BlockSpec((BN, D), lambda r, n: (r * NB + n, 0)),
        out_shape=jax.ShapeDtypeStruct((RN, D), jnp.float32),
    )(act, W)


def _combine_body(relu, a0_ref, a1_ref, act_ref, root_ref, b_ref, o_ref):
    h = (a0_ref[...] + a1_ref[...] + b_ref[...]
         + jnp.dot(act_ref[...], root_ref[...],
                   preferred_element_type=jnp.float32,
                   precision=jax.lax.Precision.HIGHEST))
    if relu:
        h = jnp.maximum(h, 0.0)
    o_ref[...] = h


def _tc_combine(aggs, act, root, b, relu):
    return pl.pallas_call(
        functools.partial(_combine_body, relu),
        grid=(NB,),
        in_specs=[pl.BlockSpec((BN, D), lambda n: (n, 0)),
                  pl.BlockSpec((BN, D), lambda n: (NB + n, 0)),
                  pl.BlockSpec((BN, D), lambda n: (n, 0)),
                  pl.BlockSpec((D, D), lambda n: (0, 0)),
                  pl.BlockSpec((1, D), lambda n: (0, 0))],
        out_specs=pl.BlockSpec((BN, D), lambda n: (n, 0)),
        out_shape=jax.ShapeDtypeStruct((N, D), jnp.float32),
    )(aggs, aggs, act, root, b.reshape(1, D))


def kernel(x, edge_index, edge_type, W1, root1, b1, W2, root2, b2):
    src = edge_index[0]
    dst = edge_index[1]
    gidx, scale, _ = _prep(edge_type, src, dst)
    gidx2 = gidx.reshape(E // CHE, CHE)
    scale = scale.reshape(E)
    dst2 = dst.reshape(E // CHE, CHE)
    h = x
    for (W, root, b, relu) in ((W1, root1, b1, True), (W2, root2, b2, False)):
        xw = _tc_xw(h, W)
        aggs = _edge_pass(xw, gidx2, scale, dst2)
        h = _tc_combine(aggs, h, root, b, relu)
    return h


# speedup vs baseline: 22.6024x; 1.0315x over previous
import functools

import jax
import jax.numpy as jnp
from jax import lax
from jax.experimental import pallas as pl
from jax.experimental.pallas import tpu as pltpu
from jax.experimental.pallas import tpu_sc as plsc

N = 10000
E = 320000
D = 128
R = 8
RN = R * N
NC = 2
NS = 16
NW = NC * NS
CH = 80
EC = E // NW
ECC = E // NS
NPT = N // NS
CPT = RN // NS
ZR = 125
NB = 10
BN = N // NB

_mesh = plsc.VectorSubcoreMesh(
    core_axis_name="c", subcore_axis_name="s", num_cores=NC, num_subcores=NS)


NCHO = EC // CH
CE = 2000
NSC = E // CE


@functools.partial(
    pl.kernel,
    out_type=(jax.ShapeDtypeStruct((E // CH, CH), jnp.int32),
              jax.ShapeDtypeStruct((E // CH, CH), jnp.float32),
              jax.ShapeDtypeStruct((RN,), jnp.float32)),
    mesh=_mesh,
    compiler_params=pltpu.CompilerParams(needs_layout_passes=False, use_tc_tiling_on_sc=False),
    scratch_types=[
        pltpu.VMEM((CPT + 16,), jnp.float32),
        pltpu.VMEM((CE,), jnp.int32),
        pltpu.VMEM((CE,), jnp.int32),
        pltpu.VMEM((CE,), jnp.int32),
        pltpu.VMEM((CE,), jnp.int32),
        pltpu.VMEM((EC,), jnp.int32),
        pltpu.VMEM((EC,), jnp.int32),
        pltpu.VMEM((EC,), jnp.int32),
        pltpu.VMEM((NCHO, CH), jnp.int32),
        pltpu.VMEM((NCHO, CH), jnp.float32),
        pltpu.SemaphoreType.DMA,
        pltpu.SemaphoreType.DMA,
        pltpu.SemaphoreType.DMA,
    ],
)
def _prep(et_h, src_h, dst_h, gidx_h, scale_h, inv_h,
          hist, etc0, etc1, dstc0, dstc1, etv, tmpv, sidxv, gidx2, scale2,
          sp0, sp1, sg_sem):
    c = lax.axis_index("c")
    s = lax.axis_index("s")
    w = s * NC + c
    lo = s * CPT

    def zbody(i, carry):
        hist[pl.ds(i * 16, 16)] = jnp.zeros((16,), jnp.float32)
        return carry

    lax.fori_loop(0, (CPT + 15) // 16, zbody, 0)

    obase = w * EC
    pltpu.sync_copy(et_h.at[pl.ds(obase, EC)], etv)
    pltpu.sync_copy(dst_h.at[pl.ds(obase, EC)], tmpv)

    ones16 = jnp.ones((16,), jnp.float32)
    bufs = ((etc0, dstc0, sp0), (etc1, dstc1, sp1))

    pltpu.async_copy(et_h.at[pl.ds(0, CE)], etc0, sp0)
    pltpu.async_copy(dst_h.at[pl.ds(0, CE)], dstc0, sp0)

    def scan(i, carry):
        for b in range(2):
            etc, dstc, sp = bufs[b]
            etc_o, dstc_o, sp_o = bufs[1 - b]
            k = i * 2 + b

            @pl.when(k + 1 < NSC)
            def _():
                off = (k + 1) * CE
                pltpu.async_copy(et_h.at[pl.ds(off, CE)], etc_o, sp_o)
                pltpu.async_copy(dst_h.at[pl.ds(off, CE)], dstc_o, sp_o)

            off = k * CE
            pltpu.make_async_copy(et_h.at[pl.ds(off, CE)], etc, sp).wait()
            pltpu.make_async_copy(dst_h.at[pl.ds(off, CE)], dstc, sp).wait()

            def hbody(j, hcarry):
                sl = pl.ds(j * 16, 16)
                sidx = etc[sl] * N + dstc[sl]
                li = sidx - lo
                m = (li >= 0) & (li < CPT)
                li = jnp.minimum(jnp.maximum(li, 0), CPT - 1)
                plsc.addupdate_scatter(hist, [li], ones16, mask=m)
                return hcarry

            lax.fori_loop(0, CE // 16, hbody, 0)
        return carry

    lax.fori_loop(0, NSC // 2, scan, 0)

    def ibody(i, carry):
        sl = pl.ds(i * 16, 16)
        hist[sl] = 1.0 / jnp.maximum(hist[sl], 1.0)
        return carry

    lax.fori_loop(0, (CPT + 15) // 16, ibody, 0)
    pltpu.sync_copy(hist.at[pl.ds(0, CPT)], inv_h.at[pl.ds(lo, CPT)])

    def sbody(r, carry):
        for i in range(CH // 16):
            sl = pl.ds(r * CH + i * 16, 16)
            sidxv[sl] = etv[sl] * N + tmpv[sl]
        return carry

    lax.fori_loop(0, NCHO, sbody, 0)

    pltpu.sync_copy(src_h.at[pl.ds(obase, EC)], tmpv)

    def gbody(r, carry):
        for i in range(CH // 16):
            sl = pl.ds(r * CH + i * 16, 16)
            gidx2[r, pl.ds(i * 16, 16)] = etv[sl] * N + tmpv[sl]
        return carry

    lax.fori_loop(0, NCHO, gbody, 0)
    plsc.subcore_barrier()

    def gfire(k, carry):
        idx = sidxv.at[pl.ds(k * CH, CH)]
        pltpu.async_copy(inv_h.at[idx], scale2.at[k], sg_sem)
        return carry

    lax.fori_loop(0, NCHO, gfire, 0)

    def gdrain(k, carry):
        idx = sidxv.at[pl.ds(k * CH, CH)]
        pltpu.make_async_copy(inv_h.at[idx], scale2.at[k], sg_sem).wait()
        return carry

    lax.fori_loop(0, NCHO, gdrain, 0)
    pltpu.sync_copy(gidx2, gidx_h.at[pl.ds(w * NCHO, NCHO)])
    pltpu.sync_copy(scale2, scale_h.at[pl.ds(w * NCHO, NCHO)])


CHE = 100
NCHE = EC // CHE


@functools.partial(
    pl.kernel,
    out_type=jax.ShapeDtypeStruct((NC * N, D), jnp.float32),
    mesh=_mesh,
    compiler_params=pltpu.CompilerParams(needs_layout_passes=False, use_tc_tiling_on_sc=False),
    scratch_types=[
        pltpu.VMEM_SHARED((N, D), jnp.float32),
        pltpu.VMEM((25, D), jnp.float32),
        pltpu.VMEM((NCHE, CHE), jnp.int32),
        pltpu.VMEM((EC,), jnp.float32),
        pltpu.VMEM((CHE,), jnp.int32),
        pltpu.VMEM((CHE,), jnp.int32),
        pltpu.VMEM((CHE, D), jnp.float32),
        pltpu.VMEM((CHE, D), jnp.float32),
        pltpu.SemaphoreType.DMA,
        pltpu.SemaphoreType.DMA,
        pltpu.SemaphoreType.DMA,
        pltpu.SemaphoreType.DMA,
        pltpu.SemaphoreType.DMA,
        pltpu.SemaphoreType.DMA,
        pltpu.SemaphoreType.DMA,
    ],
)
def _edge_pass(xw_h, gidx2_h, scale_h, dst2_h, aggs_h,
               acc_sp, zrows, gidxv, scalev, dstb0, dstb1,
               rows0, rows1, sz, sg0, sg1, ss0, ss1, sd0, sd1):
    c = lax.axis_index("c")
    s = lax.axis_index("s")
    w = s * NC + c

    pltpu.sync_copy(gidx2_h.at[pl.ds(w * NCHE, NCHE)], gidxv)
    pltpu.sync_copy(scale_h.at[pl.ds(w * EC, EC)], scalev)

    def zfill(r, carry):
        for i in range(D // 16):
            zrows[r, pl.ds(i * 16, 16)] = jnp.zeros((16,), jnp.float32)
        return carry

    lax.fori_loop(0, 25, zfill, 0)
    for m in range(NPT // 25):
        pltpu.async_copy(zrows, acc_sp.at[pl.ds(s * NPT + m * 25, 25)], sz)
    for m in range(NPT // 25):
        pltpu.make_async_copy(zrows, acc_sp.at[pl.ds(s * NPT + m * 25, 25)],
                              sz).wait()
    plsc.subcore_barrier()

    bufs = ((rows0, sg0, ss0, dstb0, sd0), (rows1, sg1, ss1, dstb1, sd1))
    row0 = w * NCHE

    pltpu.async_copy(xw_h.at[gidxv.at[0]], rows0, sg0)
    pltpu.async_copy(dst2_h.at[row0], dstb0, sd0)

    def pipe(i, carry):
        for b in range(2):
            rows, sg, ss, dstb, sd = bufs[b]
            rows_o, sg_o, ss_o, dstb_o, sd_o = bufs[1 - b]
            k = i * 2 + b

            @pl.when(k >= 1)
            def _():
                pltpu.make_async_copy(
                    rows_o, acc_sp.at[dstb_o], ss_o).wait()

            @pl.when(k + 1 < NCHE)
            def _():
                pltpu.async_copy(xw_h.at[gidxv.at[k + 1]], rows_o, sg_o)
                pltpu.async_copy(dst2_h.at[row0 + k + 1], dstb_o, sd_o)

            pltpu.make_async_copy(xw_h.at[gidxv.at[k]], rows, sg).wait()
            pltpu.make_async_copy(dst2_h.at[row0 + k], dstb, sd).wait()

            def mbody(j2, mcarry):
                for u in range(2):
                    j = j2 * 2 + u
                    sp = plsc.load_gather(
                        scalev, [jnp.full((16,), k * CHE + j, jnp.int32)])
                    for i2 in range(D // 16):
                        sl = pl.ds(i2 * 16, 16)
                        rows[j, sl] = rows[j, sl] * sp
                return mcarry

            lax.fori_loop(0, CHE // 2, mbody, 0)
            pltpu.async_copy(rows, acc_sp.at[dstb], ss, add=True)
        return carry

    lax.fori_loop(0, NCHE // 2, pipe, 0)
    pltpu.make_async_copy(rows1, acc_sp.at[dstb1], ss1).wait()
    plsc.subcore_barrier()
    pltpu.sync_copy(acc_sp.at[pl.ds(s * NPT, NPT)],
                    aggs_h.at[pl.ds(c * N + s * NPT, NPT)])


def _xw_body(x_ref, w_ref, o_ref):
    o_ref[...] = jnp.dot(x_ref[...], w_ref[0],
                         preferred_element_type=jnp.float32,
                         precision=jax.lax.Precision.HIGHEST)


def _tc_xw(act, W):
    return pl.pallas_call(
        _xw_body,
        grid=(R, NB),
        in_specs=[pl.BlockSpec((BN, D), lambda r, n: (n, 0)),
                  pl.BlockSpec((1, D, D), lambda r, n: (r, 0, 0))],
        out_specs=pl.BlockSpec((BN, D), lambda r, n: (r * NB + n, 0)),
        out_shape=jax.ShapeDtypeStruct((RN, D), jnp.float32),
    )(act, W)


def _combine_body(relu, a0_ref, a1_ref, act_ref, root_ref, b_ref, o_ref):
    h = (a0_ref[...] + a1_ref[...] + b_ref[...]
         + jnp.dot(act_ref[...], root_ref[...],
                   preferred_element_type=jnp.float32))
    if relu:
        h = jnp.maximum(h, 0.0)
    o_ref[...] = h


def _tc_combine(aggs, act, root, b, relu):
    return pl.pallas_call(
        functools.partial(_combine_body, relu),
        grid=(NB,),
        in_specs=[pl.BlockSpec((BN, D), lambda n: (n, 0)),
                  pl.BlockSpec((BN, D), lambda n: (NB + n, 0)),
                  pl.BlockSpec((BN, D), lambda n: (n, 0)),
                  pl.BlockSpec((D, D), lambda n: (0, 0)),
                  pl.BlockSpec((1, D), lambda n: (0, 0))],
        out_specs=pl.BlockSpec((BN, D), lambda n: (n, 0)),
        out_shape=jax.ShapeDtypeStruct((N, D), jnp.float32),
    )(aggs, aggs, act, root, b.reshape(1, D))


def kernel(x, edge_index, edge_type, W1, root1, b1, W2, root2, b2):
    src = edge_index[0]
    dst = edge_index[1]
    gidx, scale, _ = _prep(edge_type, src, dst)
    gidx2 = gidx.reshape(E // CHE, CHE)
    scale = scale.reshape(E)
    dst2 = dst.reshape(E // CHE, CHE)
    h = x
    for (W, root, b, relu) in ((W1, root1, b1, True), (W2, root2, b2, False)):
        xw = _tc_xw(h, W)
        aggs = _edge_pass(xw, gidx2, scale, dst2)
        h = _tc_combine(aggs, h, root, b, relu)
    return h


# speedup vs baseline: 23.1427x; 1.0239x over previous
import functools

import jax
import jax.numpy as jnp
from jax import lax
from jax.experimental import pallas as pl
from jax.experimental.pallas import tpu as pltpu
from jax.experimental.pallas import tpu_sc as plsc

N = 10000
E = 320000
D = 128
R = 8
RN = R * N
NC = 2
NS = 16
NW = NC * NS
CH = 80
EC = E // NW
ECC = E // NS
NPT = N // NS
CPT = RN // NS
ZR = 125
NB = 10
BN = N // NB

_mesh = plsc.VectorSubcoreMesh(
    core_axis_name="c", subcore_axis_name="s", num_cores=NC, num_subcores=NS)


NCHO = EC // CH
CE = 2000
NSC = E // CE


@functools.partial(
    pl.kernel,
    out_type=(jax.ShapeDtypeStruct((E // CH, CH), jnp.int32),
              jax.ShapeDtypeStruct((E // CH, CH), jnp.float32),
              jax.ShapeDtypeStruct((RN,), jnp.float32)),
    mesh=_mesh,
    compiler_params=pltpu.CompilerParams(needs_layout_passes=False, use_tc_tiling_on_sc=False),
    scratch_types=[
        pltpu.VMEM((CPT + 16,), jnp.float32),
        pltpu.VMEM((CE,), jnp.int32),
        pltpu.VMEM((CE,), jnp.int32),
        pltpu.VMEM((CE,), jnp.int32),
        pltpu.VMEM((CE,), jnp.int32),
        pltpu.VMEM((EC,), jnp.int32),
        pltpu.VMEM((EC,), jnp.int32),
        pltpu.VMEM((EC,), jnp.int32),
        pltpu.VMEM((NCHO, CH), jnp.int32),
        pltpu.VMEM((NCHO, CH), jnp.float32),
        pltpu.SemaphoreType.DMA,
        pltpu.SemaphoreType.DMA,
        pltpu.SemaphoreType.DMA,
    ],
)
def _prep(et_h, src_h, dst_h, gidx_h, scale_h, inv_h,
          hist, etc0, etc1, dstc0, dstc1, etv, tmpv, sidxv, gidx2, scale2,
          sp0, sp1, sg_sem):
    c = lax.axis_index("c")
    s = lax.axis_index("s")
    w = s * NC + c
    lo = s * CPT

    def zbody(i, carry):
        hist[pl.ds(i * 16, 16)] = jnp.zeros((16,), jnp.float32)
        return carry

    lax.fori_loop(0, (CPT + 15) // 16, zbody, 0)

    obase = w * EC
    pltpu.sync_copy(et_h.at[pl.ds(obase, EC)], etv)
    pltpu.sync_copy(dst_h.at[pl.ds(obase, EC)], tmpv)

    ones16 = jnp.ones((16,), jnp.float32)
    bufs = ((etc0, dstc0, sp0), (etc1, dstc1, sp1))

    pltpu.async_copy(et_h.at[pl.ds(0, CE)], etc0, sp0)
    pltpu.async_copy(dst_h.at[pl.ds(0, CE)], dstc0, sp0)

    def scan(i, carry):
        for b in range(2):
            etc, dstc, sp = bufs[b]
            etc_o, dstc_o, sp_o = bufs[1 - b]
            k = i * 2 + b

            @pl.when(k + 1 < NSC)
            def _():
                off = (k + 1) * CE
                pltpu.async_copy(et_h.at[pl.ds(off, CE)], etc_o, sp_o)
                pltpu.async_copy(dst_h.at[pl.ds(off, CE)], dstc_o, sp_o)

            off = k * CE
            pltpu.make_async_copy(et_h.at[pl.ds(off, CE)], etc, sp).wait()
            pltpu.make_async_copy(dst_h.at[pl.ds(off, CE)], dstc, sp).wait()

            def hbody(j5, hcarry):
                for u in range(5):
                    sl = pl.ds((j5 * 5 + u) * 16, 16)
                    sidx = etc[sl] * N + dstc[sl]
                    li = sidx - lo
                    m = (li >= 0) & (li < CPT)
                    li = jnp.minimum(jnp.maximum(li, 0), CPT - 1)
                    plsc.addupdate_scatter(hist, [li], ones16, mask=m)
                return hcarry

            lax.fori_loop(0, CE // 80, hbody, 0)
        return carry

    lax.fori_loop(0, NSC // 2, scan, 0)

    def ibody(i, carry):
        sl = pl.ds(i * 16, 16)
        hist[sl] = 1.0 / jnp.maximum(hist[sl], 1.0)
        return carry

    lax.fori_loop(0, (CPT + 15) // 16, ibody, 0)
    pltpu.sync_copy(hist.at[pl.ds(0, CPT)], inv_h.at[pl.ds(lo, CPT)])

    def sbody(r, carry):
        for i in range(CH // 16):
            sl = pl.ds(r * CH + i * 16, 16)
            sidxv[sl] = etv[sl] * N + tmpv[sl]
        return carry

    lax.fori_loop(0, NCHO, sbody, 0)

    pltpu.sync_copy(src_h.at[pl.ds(obase, EC)], tmpv)

    def gbody(r, carry):
        for i in range(CH // 16):
            sl = pl.ds(r * CH + i * 16, 16)
            gidx2[r, pl.ds(i * 16, 16)] = etv[sl] * N + tmpv[sl]
        return carry

    lax.fori_loop(0, NCHO, gbody, 0)
    plsc.subcore_barrier()

    def gfire(k, carry):
        idx = sidxv.at[pl.ds(k * CH, CH)]
        pltpu.async_copy(inv_h.at[idx], scale2.at[k], sg_sem)
        return carry

    lax.fori_loop(0, NCHO, gfire, 0)

    def gdrain(k, carry):
        idx = sidxv.at[pl.ds(k * CH, CH)]
        pltpu.make_async_copy(inv_h.at[idx], scale2.at[k], sg_sem).wait()
        return carry

    lax.fori_loop(0, NCHO, gdrain, 0)
    pltpu.sync_copy(gidx2, gidx_h.at[pl.ds(w * NCHO, NCHO)])
    pltpu.sync_copy(scale2, scale_h.at[pl.ds(w * NCHO, NCHO)])


CHE = 100
NCHE = EC // CHE


@functools.partial(
    pl.kernel,
    out_type=jax.ShapeDtypeStruct((NC * N, D), jnp.float32),
    mesh=_mesh,
    compiler_params=pltpu.CompilerParams(needs_layout_passes=False, use_tc_tiling_on_sc=False),
    scratch_types=[
        pltpu.VMEM_SHARED((N, D), jnp.float32),
        pltpu.VMEM((25, D), jnp.float32),
        pltpu.VMEM((NCHE, CHE), jnp.int32),
        pltpu.VMEM((EC,), jnp.float32),
        pltpu.VMEM((CHE,), jnp.int32),
        pltpu.VMEM((CHE,), jnp.int32),
        pltpu.VMEM((CHE, D), jnp.float32),
        pltpu.VMEM((CHE, D), jnp.float32),
        pltpu.SemaphoreType.DMA,
        pltpu.SemaphoreType.DMA,
        pltpu.SemaphoreType.DMA,
        pltpu.SemaphoreType.DMA,
        pltpu.SemaphoreType.DMA,
        pltpu.SemaphoreType.DMA,
        pltpu.SemaphoreType.DMA,
    ],
)
def _edge_pass(xw_h, gidx2_h, scale_h, dst2_h, aggs_h,
               acc_sp, zrows, gidxv, scalev, dstb0, dstb1,
               rows0, rows1, sz, sg0, sg1, ss0, ss1, sd0, sd1):
    c = lax.axis_index("c")
    s = lax.axis_index("s")
    w = s * NC + c

    pltpu.sync_copy(gidx2_h.at[pl.ds(w * NCHE, NCHE)], gidxv)
    pltpu.sync_copy(scale_h.at[pl.ds(w * EC, EC)], scalev)

    def zfill(r, carry):
        for i in range(D // 16):
            zrows[r, pl.ds(i * 16, 16)] = jnp.zeros((16,), jnp.float32)
        return carry

    lax.fori_loop(0, 25, zfill, 0)
    for m in range(NPT // 25):
        pltpu.async_copy(zrows, acc_sp.at[pl.ds(s * NPT + m * 25, 25)], sz)
    for m in range(NPT // 25):
        pltpu.make_async_copy(zrows, acc_sp.at[pl.ds(s * NPT + m * 25, 25)],
                              sz).wait()
    plsc.subcore_barrier()

    bufs = ((rows0, sg0, ss0, dstb0, sd0), (rows1, sg1, ss1, dstb1, sd1))
    row0 = w * NCHE

    pltpu.async_copy(xw_h.at[gidxv.at[0]], rows0, sg0)
    pltpu.async_copy(dst2_h.at[row0], dstb0, sd0)

    def pipe(i, carry):
        for b in range(2):
            rows, sg, ss, dstb, sd = bufs[b]
            rows_o, sg_o, ss_o, dstb_o, sd_o = bufs[1 - b]
            k = i * 2 + b

            @pl.when(k >= 1)
            def _():
                pltpu.make_async_copy(
                    rows_o, acc_sp.at[dstb_o], ss_o).wait()

            @pl.when(k + 1 < NCHE)
            def _():
                pltpu.async_copy(xw_h.at[gidxv.at[k + 1]], rows_o, sg_o)
                pltpu.async_copy(dst2_h.at[row0 + k + 1], dstb_o, sd_o)

            pltpu.make_async_copy(xw_h.at[gidxv.at[k]], rows, sg).wait()
            pltpu.make_async_copy(dst2_h.at[row0 + k], dstb, sd).wait()

            def mbody(j2, mcarry):
                for u in range(4):
                    j = j2 * 4 + u
                    sp = plsc.load_gather(
                        scalev, [jnp.full((16,), k * CHE + j, jnp.int32)])
                    for i2 in range(D // 16):
                        sl = pl.ds(i2 * 16, 16)
                        rows[j, sl] = rows[j, sl] * sp
                return mcarry

            lax.fori_loop(0, CHE // 4, mbody, 0)
            pltpu.async_copy(rows, acc_sp.at[dstb], ss, add=True)
        return carry

    lax.fori_loop(0, NCHE // 2, pipe, 0)
    pltpu.make_async_copy(rows1, acc_sp.at[dstb1], ss1).wait()
    plsc.subcore_barrier()
    pltpu.sync_copy(acc_sp.at[pl.ds(s * NPT, NPT)],
                    aggs_h.at[pl.ds(c * N + s * NPT, NPT)])


def _xw_body(x_ref, w_ref, o_ref):
    o_ref[...] = jnp.dot(x_ref[...], w_ref[0],
                         preferred_element_type=jnp.float32,
                         precision=jax.lax.Precision.HIGHEST)


def _tc_xw(act, W):
    return pl.pallas_call(
        _xw_body,
        grid=(R, NB),
        in_specs=[pl.BlockSpec((BN, D), lambda r, n: (n, 0)),
                  pl.BlockSpec((1, D, D), lambda r, n: (r, 0, 0))],
        out_specs=pl.BlockSpec((BN, D), lambda r, n: (r * NB + n, 0)),
        out_shape=jax.ShapeDtypeStruct((RN, D), jnp.float32),
    )(act, W)


def _combine_body(relu, a0_ref, a1_ref, act_ref, root_ref, b_ref, o_ref):
    h = (a0_ref[...] + a1_ref[...] + b_ref[...]
         + jnp.dot(act_ref[...], root_ref[...],
                   preferred_element_type=jnp.float32))
    if relu:
        h = jnp.maximum(h, 0.0)
    o_ref[...] = h


def _tc_combine(aggs, act, root, b, relu):
    return pl.pallas_call(
        functools.partial(_combine_body, relu),
        grid=(NB,),
        in_specs=[pl.BlockSpec((BN, D), lambda n: (n, 0)),
                  pl.BlockSpec((BN, D), lambda n: (NB + n, 0)),
                  pl.BlockSpec((BN, D), lambda n: (n, 0)),
                  pl.BlockSpec((D, D), lambda n: (0, 0)),
                  pl.BlockSpec((1, D), lambda n: (0, 0))],
        out_specs=pl.BlockSpec((BN, D), lambda n: (n, 0)),
        out_shape=jax.ShapeDtypeStruct((N, D), jnp.float32),
    )(aggs, aggs, act, root, b.reshape(1, D))


def kernel(x, edge_index, edge_type, W1, root1, b1, W2, root2, b2):
    src = edge_index[0]
    dst = edge_index[1]
    gidx, scale, _ = _prep(edge_type, src, dst)
    gidx2 = gidx.reshape(E // CHE, CHE)
    scale = scale.reshape(E)
    dst2 = dst.reshape(E // CHE, CHE)
    h = x
    for (W, root, b, relu) in ((W1, root1, b1, True), (W2, root2, b2, False)):
        xw = _tc_xw(h, W)
        aggs = _edge_pass(xw, gidx2, scale, dst2)
        h = _tc_combine(aggs, h, root, b, relu)
    return h
